# Initial kernel scaffold; baseline (speedup 1.0000x reference)
import jax
import jax.numpy as jnp
from jax.experimental import pallas as pl


def kernel(x, W1, b1, W2, b2, W3, b3, codebook):
    raise NotImplementedError("write your pallas kernel here")


# speedup vs baseline: 1.5888x; 1.5888x over previous
import jax
import jax.numpy as jnp
from jax import lax
from jax.experimental import pallas as pl


def _fused_kernel(split, split_dim, x_ref, w1_ref, b1_ref, w2_ref, b2_ref,
                  w3_ref, b3_ref, ct_ref, z_ref, idx_ref):
    x = x_ref[...]
    h = jnp.dot(x, w1_ref[...], preferred_element_type=jnp.float32) + b1_ref[...]
    h = jnp.where(h >= 0, h, 0.2 * h)
    h = jnp.dot(h, w2_ref[...], preferred_element_type=jnp.float32) + b2_ref[...]
    h = jnp.where(h >= 0, h, 0.2 * h)
    z = jnp.dot(h, w3_ref[...], preferred_element_type=jnp.float32) + b3_ref[...]
    z_ref[...] = z

    ct = ct_ref[...]
    k = ct.shape[1]
    code_sq = jnp.sum(ct * ct, axis=0, keepdims=True)
    idx_rows = []
    for j in range(split):
        v = z[:, j * split_dim:(j + 1) * split_dim]
        v_sq = jnp.sum(v * v, axis=1, keepdims=True)
        d2 = v_sq + code_sq - 2.0 * jnp.dot(v, ct, preferred_element_type=jnp.float32)
        m = jnp.min(d2, axis=1, keepdims=True)
        iota = lax.broadcasted_iota(jnp.int32, d2.shape, 1)
        idx = jnp.min(jnp.where(d2 == m, iota, k), axis=1)
        idx_rows.append(idx)
    idx_ref[...] = jnp.stack(idx_rows, axis=0)


def kernel(x, W1, b1, W2, b2, W3, b3, codebook):
    n, input_dim = x.shape
    d1 = W1.shape[1]
    d2 = W2.shape[1]
    z_dim = W3.shape[1]
    k, split_dim = codebook.shape
    split = z_dim // split_dim

    bn = 512
    n_blocks = n // bn

    ct = codebook.T
    b1r = b1.reshape(1, d1)
    b2r = b2.reshape(1, d2)
    b3r = b3.reshape(1, z_dim)

    import functools
    body = functools.partial(_fused_kernel, split, split_dim)
    z, idxs = pl.pallas_call(
        body,
        grid=(n_blocks,),
        in_specs=[
            pl.BlockSpec((bn, input_dim), lambda i: (i, 0)),
            pl.BlockSpec((input_dim, d1), lambda i: (0, 0)),
            pl.BlockSpec((1, d1), lambda i: (0, 0)),
            pl.BlockSpec((d1, d2), lambda i: (0, 0)),
            pl.BlockSpec((1, d2), lambda i: (0, 0)),
            pl.BlockSpec((d2, z_dim), lambda i: (0, 0)),
            pl.BlockSpec((1, z_dim), lambda i: (0, 0)),
            pl.BlockSpec((split_dim, k), lambda i: (0, 0)),
        ],
        out_specs=[
            pl.BlockSpec((bn, z_dim), lambda i: (i, 0)),
            pl.BlockSpec((split, bn), lambda i: (0, i)),
        ],
        out_shape=[
            jax.ShapeDtypeStruct((n, z_dim), jnp.float32),
            jax.ShapeDtypeStruct((split, n), jnp.int32),
        ],
    )(x, W1, b1r, W2, b2r, W3, b3r, ct)

    indices = idxs.T.astype(jnp.int64)
    return (z, indices)


# speedup vs baseline: 1.6623x; 1.0463x over previous
import jax
import jax.numpy as jnp
from jax import lax
from jax.experimental import pallas as pl


def _fused_kernel(split, split_dim, x_ref, w1_ref, b1_ref, w2_ref, b2_ref,
                  w3_ref, b3_ref, ct_ref, z_ref, idx_ref):
    x = x_ref[...]
    h = jnp.dot(x, w1_ref[...], preferred_element_type=jnp.float32) + b1_ref[...]
    h = jnp.where(h >= 0, h, 0.2 * h)
    h = jnp.dot(h, w2_ref[...], preferred_element_type=jnp.float32) + b2_ref[...]
    h = jnp.where(h >= 0, h, 0.2 * h)
    z = jnp.dot(h, w3_ref[...], preferred_element_type=jnp.float32) + b3_ref[...]
    z_ref[...] = z

    ct = ct_ref[...]
    k = ct.shape[1]
    code_sq = jnp.sum(ct * ct, axis=0, keepdims=True)
    ct_m2 = -2.0 * ct
    idx_rows = []
    for j in range(split):
        v = z[:, j * split_dim:(j + 1) * split_dim]
        s = jnp.dot(v, ct_m2, preferred_element_type=jnp.float32) + code_sq
        m = jnp.min(s, axis=1, keepdims=True)
        iota = lax.broadcasted_iota(jnp.int32, s.shape, 1)
        idx = jnp.min(jnp.where(s == m, iota, k), axis=1)
        idx_rows.append(idx)
    idx_ref[...] = jnp.stack(idx_rows, axis=0)


def kernel(x, W1, b1, W2, b2, W3, b3, codebook):
    n, input_dim = x.shape
    d1 = W1.shape[1]
    d2 = W2.shape[1]
    z_dim = W3.shape[1]
    k, split_dim = codebook.shape
    split = z_dim // split_dim

    bn = 512
    n_blocks = n // bn

    ct = codebook.T
    b1r = b1.reshape(1, d1)
    b2r = b2.reshape(1, d2)
    b3r = b3.reshape(1, z_dim)

    import functools
    body = functools.partial(_fused_kernel, split, split_dim)
    z, idxs = pl.pallas_call(
        body,
        grid=(n_blocks,),
        in_specs=[
            pl.BlockSpec((bn, input_dim), lambda i: (i, 0)),
            pl.BlockSpec((input_dim, d1), lambda i: (0, 0)),
            pl.BlockSpec((1, d1), lambda i: (0, 0)),
            pl.BlockSpec((d1, d2), lambda i: (0, 0)),
            pl.BlockSpec((1, d2), lambda i: (0, 0)),
            pl.BlockSpec((d2, z_dim), lambda i: (0, 0)),
            pl.BlockSpec((1, z_dim), lambda i: (0, 0)),
            pl.BlockSpec((split_dim, k), lambda i: (0, 0)),
        ],
        out_specs=[
            pl.BlockSpec((bn, z_dim), lambda i: (i, 0)),
            pl.BlockSpec((split, bn), lambda i: (0, i)),
        ],
        out_shape=[
            jax.ShapeDtypeStruct((n, z_dim), jnp.float32),
            jax.ShapeDtypeStruct((split, n), jnp.int32),
        ],
    )(x, W1, b1r, W2, b2r, W3, b3r, ct)

    indices = idxs.T.astype(jnp.int64)
    return (z, indices)


# speedup vs baseline: 2.1791x; 1.3109x over previous
import jax
import jax.numpy as jnp
from jax import lax
from jax.experimental import pallas as pl


def _fused_kernel(split, split_dim, x_ref, w1_ref, b1_ref, w2_ref, b2_ref,
                  w3_ref, b3_ref, ct_ref, z_ref, idx_ref):
    x = x_ref[...]
    h = jnp.dot(x, w1_ref[...], preferred_element_type=jnp.float32) + b1_ref[...]
    h = jnp.where(h >= 0, h, 0.2 * h)
    h = jnp.dot(h, w2_ref[...], preferred_element_type=jnp.float32) + b2_ref[...]
    h = jnp.where(h >= 0, h, 0.2 * h)
    z = jnp.dot(h, w3_ref[...], preferred_element_type=jnp.float32) + b3_ref[...]
    z_ref[...] = z

    ct = ct_ref[...]
    k = ct.shape[1]
    bn = z.shape[0]
    code_sq = jnp.sum(ct * ct, axis=0, keepdims=True)
    ct_aug = jnp.concatenate([-2.0 * ct, code_sq], axis=0)
    ones_col = jnp.ones((bn, 1), dtype=jnp.float32)

    nc = k // 128
    iota_l = lax.broadcasted_iota(jnp.int32, (bn, 128), 1).astype(jnp.float32)
    idx_rows = []
    for j in range(split):
        v = z[:, j * split_dim:(j + 1) * split_dim]
        va = jnp.concatenate([v, ones_col], axis=1)
        s = jnp.dot(va, ct_aug, preferred_element_type=jnp.float32)
        m1 = s[:, 0:128]
        for c in range(1, nc):
            m1 = jnp.minimum(m1, s[:, c * 128:(c + 1) * 128])
        c1 = jnp.full((bn, 128), float(nc), dtype=jnp.float32)
        for c in range(nc):
            hit = s[:, c * 128:(c + 1) * 128] == m1
            c1 = jnp.minimum(c1, jnp.where(hit, float(c), float(nc)))
        m = jnp.min(m1, axis=1, keepdims=True)
        k_l = c1 * 128.0 + iota_l
        idx_f = jnp.min(jnp.where(m1 == m, k_l, float(2 * k)), axis=1)
        idx_rows.append(idx_f)
    idx_ref[...] = jnp.stack(idx_rows, axis=0).astype(jnp.int32)


def kernel(x, W1, b1, W2, b2, W3, b3, codebook):
    n, input_dim = x.shape
    d1 = W1.shape[1]
    d2 = W2.shape[1]
    z_dim = W3.shape[1]
    k, split_dim = codebook.shape
    split = z_dim // split_dim

    bn = 512
    n_blocks = n // bn

    ct = codebook.T
    b1r = b1.reshape(1, d1)
    b2r = b2.reshape(1, d2)
    b3r = b3.reshape(1, z_dim)

    import functools
    body = functools.partial(_fused_kernel, split, split_dim)
    z, idxs = pl.pallas_call(
        body,
        grid=(n_blocks,),
        in_specs=[
            pl.BlockSpec((bn, input_dim), lambda i: (i, 0)),
            pl.BlockSpec((input_dim, d1), lambda i: (0, 0)),
            pl.BlockSpec((1, d1), lambda i: (0, 0)),
            pl.BlockSpec((d1, d2), lambda i: (0, 0)),
            pl.BlockSpec((1, d2), lambda i: (0, 0)),
            pl.BlockSpec((d2, z_dim), lambda i: (0, 0)),
            pl.BlockSpec((1, z_dim), lambda i: (0, 0)),
            pl.BlockSpec((split_dim, k), lambda i: (0, 0)),
        ],
        out_specs=[
            pl.BlockSpec((bn, z_dim), lambda i: (i, 0)),
            pl.BlockSpec((split, bn), lambda i: (0, i)),
        ],
        out_shape=[
            jax.ShapeDtypeStruct((n, z_dim), jnp.float32),
            jax.ShapeDtypeStruct((split, n), jnp.int32),
        ],
    )(x, W1, b1r, W2, b2r, W3, b3r, ct)

    indices = idxs.T.astype(jnp.int64)
    return (z, indices)
